# baseline (speedup 1.0000x reference)
import functools
import jax
import jax.numpy as jnp
from jax import lax
from jax.experimental import pallas as pl
from jax.experimental.pallas import tpu as pltpu
from jax.experimental.pallas import tpu_sc as plsc

E = 16
K = 2
D = 1024
F = 1024
T = 2048

BT = 256
NB = 32
S = NB * BT

NC = 2
NS = 16
NW = NC * NS
TPW = T // NW
CH = 32


def _route_body(x_ref, gw_ref, d0_ref, d1_ref, w0_ref, w1_ref, be_ref, nu_ref):
    x = x_ref[...]
    logits = jnp.dot(x, gw_ref[...], preferred_element_type=jnp.float32)
    m = jnp.max(logits, axis=-1, keepdims=True)
    ex = jnp.exp(logits - m)
    p = ex / jnp.sum(ex, axis=-1, keepdims=True)

    iota = lax.broadcasted_iota(jnp.int32, (T, E), 1)
    m0 = jnp.max(p, axis=-1, keepdims=True)
    i0 = jnp.min(jnp.where(p == m0, iota, E), axis=-1, keepdims=True)
    p_ex = jnp.where(iota == i0, -jnp.inf, p)
    m1 = jnp.max(p_ex, axis=-1, keepdims=True)
    i1 = jnp.min(jnp.where(p_ex == m1, iota, E), axis=-1, keepdims=True)
    wsum = m0 + m1
    w0_ref[...] = jnp.broadcast_to(m0 / wsum, (T, 16))
    w1_ref[...] = jnp.broadcast_to(m1 / wsum, (T, 16))

    onehot0 = (iota == i0).astype(jnp.float32)
    onehot1 = (iota == i1).astype(jnp.float32)
    cnt = onehot0 + onehot1

    n = jnp.sum(cnt, axis=0, keepdims=True)
    n_pad = jnp.floor((n + (BT - 1)) * (1.0 / BT)) * BT
    tri_e = (lax.broadcasted_iota(jnp.int32, (E, E), 0)
             < lax.broadcasted_iota(jnp.int32, (E, E), 1)).astype(jnp.float32)
    offs = jnp.dot(n_pad, tri_e, preferred_element_type=jnp.float32)
    ends = offs + n_pad

    bb = (lax.broadcasted_iota(jnp.int32, (NB, E), 0) * BT).astype(jnp.float32)
    ends_b = jnp.broadcast_to(ends, (NB, E))
    be = jnp.sum((bb >= ends_b).astype(jnp.float32), axis=-1, keepdims=True)
    be_ref[...] = jnp.minimum(be, float(E - 1)).astype(jnp.int32)
    nu_ref[...] = (jnp.sum(n_pad, axis=-1, keepdims=True)
                   * (1.0 / BT)).astype(jnp.int32)

    RC = 256
    NCK = T // RC
    t_iota = lax.broadcasted_iota(jnp.int32, (NCK, T), 1)
    c_iota = lax.broadcasted_iota(jnp.int32, (NCK, T), 0)
    sel = (t_iota // RC == c_iota).astype(jnp.float32)
    chunk_cnt = jnp.dot(sel, cnt, preferred_element_type=jnp.float32)
    tri_c = (lax.broadcasted_iota(jnp.int32, (NCK, NCK), 0)
             > lax.broadcasted_iota(jnp.int32, (NCK, NCK), 1)).astype(jnp.float32)
    excl_chunk = jnp.dot(tri_c, chunk_cnt, preferred_element_type=jnp.float32)

    row_l = lax.broadcasted_iota(jnp.int32, (RC, RC), 0)
    col_l = lax.broadcasted_iota(jnp.int32, (RC, RC), 1)
    tri_l = (col_l < row_l).astype(jnp.float32)
    for c in range(NCK):
        sl = slice(c * RC, (c + 1) * RC)
        rank = jnp.dot(tri_l, cnt[sl], preferred_element_type=jnp.float32)
        slot = jnp.broadcast_to(offs + excl_chunk[c:c + 1], (RC, E)) + rank
        d0 = jnp.sum(onehot0[sl] * slot, axis=-1, keepdims=True)
        d1 = jnp.sum(onehot1[sl] * slot, axis=-1, keepdims=True)
        d0_ref[sl] = d0.astype(jnp.int32)
        d1_ref[sl] = d1.astype(jnp.int32)


def _route(hidden_states, gate_w):
    return pl.pallas_call(
        _route_body,
        out_shape=[
            jax.ShapeDtypeStruct((T, 1), jnp.int32),
            jax.ShapeDtypeStruct((T, 1), jnp.int32),
            jax.ShapeDtypeStruct((T, 16), jnp.float32),
            jax.ShapeDtypeStruct((T, 16), jnp.float32),
            jax.ShapeDtypeStruct((NB, 1), jnp.int32),
            jax.ShapeDtypeStruct((1, 1), jnp.int32),
        ],
    )(hidden_states, gate_w)


_MESH = plsc.VectorSubcoreMesh(core_axis_name="c", subcore_axis_name="s")


@functools.partial(
    pl.kernel,
    out_type=jax.ShapeDtypeStruct((S, D), jnp.float32),
    mesh=_MESH,
    scratch_types=[
        pltpu.VMEM((TPW,), jnp.int32),
        pltpu.VMEM((TPW, D), jnp.float32),
        pltpu.SemaphoreType.DMA,
    ],
)
def _sc_scatter(hid_hbm, d0_hbm, d1_hbm, xs_hbm, idx_v, rows_v, sem):
    wid = lax.axis_index("s") * NC + lax.axis_index("c")
    base = wid * TPW
    pltpu.sync_copy(hid_hbm.at[pl.ds(base, TPW)], rows_v)
    pltpu.sync_copy(d0_hbm.at[pl.ds(base, TPW)], idx_v)
    pltpu.async_copy(rows_v, xs_hbm.at[idx_v], sem).wait()
    pltpu.sync_copy(d1_hbm.at[pl.ds(base, TPW)], idx_v)
    pltpu.async_copy(rows_v, xs_hbm.at[idx_v], sem).wait()


def _gmm_body(be_ref, nu_ref, xs_ref, wg_ref, wu_ref, wd_ref, ys_ref):
    b = pl.program_id(0)

    @pl.when(b < nu_ref[0])
    def _():
        x = xs_ref[...].astype(jnp.bfloat16)
        h = jax.nn.silu(jnp.dot(x, wg_ref[0], preferred_element_type=jnp.float32))
        h = h * jnp.dot(x, wu_ref[0], preferred_element_type=jnp.float32)
        ys_ref[...] = jnp.dot(h.astype(jnp.bfloat16), wd_ref[0],
                              preferred_element_type=jnp.float32)


def _gmm(xs, w_gate, w_up, w_down, be, nu):
    grid_spec = pltpu.PrefetchScalarGridSpec(
        num_scalar_prefetch=2,
        grid=(NB,),
        in_specs=[
            pl.BlockSpec((BT, D), lambda b, be, nu: (b, 0)),
            pl.BlockSpec((1, D, F), lambda b, be, nu: (be[b], 0, 0)),
            pl.BlockSpec((1, D, F), lambda b, be, nu: (be[b], 0, 0)),
            pl.BlockSpec((1, F, D), lambda b, be, nu: (be[b], 0, 0)),
        ],
        out_specs=pl.BlockSpec((BT, D), lambda b, be, nu: (b, 0)),
    )
    return pl.pallas_call(
        _gmm_body,
        grid_spec=grid_spec,
        out_shape=jax.ShapeDtypeStruct((S, D), jnp.float32),
    )(be, nu, xs,
      w_gate.astype(jnp.bfloat16),
      w_up.astype(jnp.bfloat16),
      w_down.astype(jnp.bfloat16))


@functools.partial(
    pl.kernel,
    out_type=jax.ShapeDtypeStruct((T, D), jnp.float32),
    mesh=_MESH,
    scratch_types=[
        pltpu.VMEM((CH,), jnp.int32),
        pltpu.VMEM((CH,), jnp.int32),
        pltpu.VMEM((CH, 16), jnp.float32),
        pltpu.VMEM((CH, 16), jnp.float32),
        pltpu.VMEM((CH, D), jnp.float32),
        pltpu.VMEM((CH, D), jnp.float32),
        pltpu.SemaphoreType.DMA,
        pltpu.SemaphoreType.DMA,
    ],
)
def _sc_combine(ys_hbm, d0_hbm, d1_hbm, w0_hbm, w1_hbm, out_hbm,
                i0_v, i1_v, w0_v, w1_v, buf0, buf1, sem0, sem1):
    wid = lax.axis_index("s") * NC + lax.axis_index("c")

    for c in range(TPW // CH):
        base = wid * TPW + c * CH
        pltpu.sync_copy(d0_hbm.at[pl.ds(base, CH)], i0_v)
        pltpu.sync_copy(d1_hbm.at[pl.ds(base, CH)], i1_v)
        cp0 = pltpu.async_copy(ys_hbm.at[i0_v], buf0, sem0)
        cp1 = pltpu.async_copy(ys_hbm.at[i1_v], buf1, sem1)
        pltpu.sync_copy(w0_hbm.at[pl.ds(base, CH)], w0_v)
        pltpu.sync_copy(w1_hbm.at[pl.ds(base, CH)], w1_v)
        cp0.wait()
        cp1.wait()

        def token_body(t, _):
            w0s = w0_v[t, :]
            w1s = w1_v[t, :]
            for v in range(D // 16):
                y0 = buf0[t, pl.ds(v * 16, 16)]
                y1 = buf1[t, pl.ds(v * 16, 16)]
                buf0[t, pl.ds(v * 16, 16)] = w0s * y0 + w1s * y1
            return 0

        lax.fori_loop(0, CH, token_body, 0)
        pltpu.sync_copy(buf0, out_hbm.at[pl.ds(base, CH)])


@jax.jit
def kernel(hidden_states, gate_w, w_gate, w_up, w_down):
    d0, d1, w0, w1, be, nu = _route(hidden_states, gate_w)
    d0r = d0.reshape(T)
    d1r = d1.reshape(T)
    xs = _sc_scatter(hidden_states, d0r, d1r)
    ys = _gmm(xs, w_gate, w_up, w_down, be.reshape(NB), nu.reshape(1))
    return _sc_combine(ys, d0r, d1r, w0, w1)


# speedup vs baseline: 1.3827x; 1.3827x over previous
import functools
import jax
import jax.numpy as jnp
from jax import lax
from jax.experimental import pallas as pl
from jax.experimental.pallas import tpu as pltpu
from jax.experimental.pallas import tpu_sc as plsc

E = 16
K = 2
D = 1024
F = 1024
T = 2048

BT = 256
NB = 32
S = NB * BT

NC = 2
NS = 16
NW = NC * NS
TPW = T // NW
CH = 32


def _route_body(x_ref, gw_ref, d0_ref, d1_ref, w0_ref, w1_ref, be_ref, nu_ref):
    x = x_ref[...]
    logits = jnp.dot(x, gw_ref[...], preferred_element_type=jnp.float32)
    m = jnp.max(logits, axis=-1, keepdims=True)
    ex = jnp.exp(logits - m)
    p = ex / jnp.sum(ex, axis=-1, keepdims=True)

    iota = lax.broadcasted_iota(jnp.int32, (T, E), 1)
    m0 = jnp.max(p, axis=-1, keepdims=True)
    i0 = jnp.min(jnp.where(p == m0, iota, E), axis=-1, keepdims=True)
    p_ex = jnp.where(iota == i0, -jnp.inf, p)
    m1 = jnp.max(p_ex, axis=-1, keepdims=True)
    i1 = jnp.min(jnp.where(p_ex == m1, iota, E), axis=-1, keepdims=True)
    wsum = m0 + m1
    w0_ref[...] = jnp.broadcast_to(m0 / wsum, (T, 16))
    w1_ref[...] = jnp.broadcast_to(m1 / wsum, (T, 16))

    onehot0 = (iota == i0).astype(jnp.float32)
    onehot1 = (iota == i1).astype(jnp.float32)
    cnt = onehot0 + onehot1

    n = jnp.sum(cnt, axis=0, keepdims=True)
    n_pad = jnp.floor((n + (BT - 1)) * (1.0 / BT)) * BT
    tri_e = (lax.broadcasted_iota(jnp.int32, (E, E), 0)
             < lax.broadcasted_iota(jnp.int32, (E, E), 1)).astype(jnp.float32)
    offs = jnp.dot(n_pad, tri_e, preferred_element_type=jnp.float32)
    ends = offs + n_pad

    bb = (lax.broadcasted_iota(jnp.int32, (NB, E), 0) * BT).astype(jnp.float32)
    ends_b = jnp.broadcast_to(ends, (NB, E))
    be = jnp.sum((bb >= ends_b).astype(jnp.float32), axis=-1, keepdims=True)
    be_ref[...] = jnp.minimum(be, float(E - 1)).astype(jnp.int32)
    nu_ref[...] = (jnp.sum(n_pad, axis=-1, keepdims=True)
                   * (1.0 / BT)).astype(jnp.int32)

    RC = 256
    NCK = T // RC
    t_iota = lax.broadcasted_iota(jnp.int32, (NCK, T), 1)
    c_iota = lax.broadcasted_iota(jnp.int32, (NCK, T), 0)
    sel = (t_iota // RC == c_iota).astype(jnp.float32)
    chunk_cnt = jnp.dot(sel, cnt, preferred_element_type=jnp.float32)
    tri_c = (lax.broadcasted_iota(jnp.int32, (NCK, NCK), 0)
             > lax.broadcasted_iota(jnp.int32, (NCK, NCK), 1)).astype(jnp.float32)
    excl_chunk = jnp.dot(tri_c, chunk_cnt, preferred_element_type=jnp.float32)

    row_l = lax.broadcasted_iota(jnp.int32, (RC, RC), 0)
    col_l = lax.broadcasted_iota(jnp.int32, (RC, RC), 1)
    tri_l = (col_l < row_l).astype(jnp.float32)
    for c in range(NCK):
        sl = slice(c * RC, (c + 1) * RC)
        rank = jnp.dot(tri_l, cnt[sl], preferred_element_type=jnp.float32)
        slot = jnp.broadcast_to(offs + excl_chunk[c:c + 1], (RC, E)) + rank
        d0 = jnp.sum(onehot0[sl] * slot, axis=-1, keepdims=True)
        d1 = jnp.sum(onehot1[sl] * slot, axis=-1, keepdims=True)
        d0_ref[sl] = d0.astype(jnp.int32)
        d1_ref[sl] = d1.astype(jnp.int32)


def _route(hidden_states, gate_w):
    return pl.pallas_call(
        _route_body,
        out_shape=[
            jax.ShapeDtypeStruct((T, 1), jnp.int32),
            jax.ShapeDtypeStruct((T, 1), jnp.int32),
            jax.ShapeDtypeStruct((T, 16), jnp.float32),
            jax.ShapeDtypeStruct((T, 16), jnp.float32),
            jax.ShapeDtypeStruct((NB, 1), jnp.int32),
            jax.ShapeDtypeStruct((1, 1), jnp.int32),
        ],
    )(hidden_states, gate_w)


_MESH = plsc.VectorSubcoreMesh(core_axis_name="c", subcore_axis_name="s")


@functools.partial(
    pl.kernel,
    out_type=jax.ShapeDtypeStruct((S, D), jnp.float32),
    mesh=_MESH,
    scratch_types=[
        pltpu.VMEM((TPW,), jnp.int32),
        pltpu.VMEM((TPW, D), jnp.float32),
        pltpu.SemaphoreType.DMA,
    ],
)
def _sc_scatter(hid_hbm, d0_hbm, d1_hbm, xs_hbm, idx_v, rows_v, sem):
    wid = lax.axis_index("s") * NC + lax.axis_index("c")
    base = wid * TPW
    pltpu.sync_copy(hid_hbm.at[pl.ds(base, TPW)], rows_v)
    pltpu.sync_copy(d0_hbm.at[pl.ds(base, TPW)], idx_v)
    pltpu.async_copy(rows_v, xs_hbm.at[idx_v], sem).wait()
    pltpu.sync_copy(d1_hbm.at[pl.ds(base, TPW)], idx_v)
    pltpu.async_copy(rows_v, xs_hbm.at[idx_v], sem).wait()


def _gmm_body(be_ref, nu_ref, xs_ref, wg_ref, wu_ref, wd_ref, ys_ref):
    b = pl.program_id(0)

    @pl.when(b < nu_ref[0])
    def _():
        x = xs_ref[...]
        h = jax.nn.silu(jnp.dot(x, wg_ref[0], preferred_element_type=jnp.float32))
        h = h * jnp.dot(x, wu_ref[0], preferred_element_type=jnp.float32)
        ys_ref[...] = jnp.dot(h, wd_ref[0], preferred_element_type=jnp.float32)


def _gmm(xs, w_gate, w_up, w_down, be, nu):
    grid_spec = pltpu.PrefetchScalarGridSpec(
        num_scalar_prefetch=2,
        grid=(NB,),
        in_specs=[
            pl.BlockSpec((BT, D), lambda b, be, nu: (b, 0)),
            pl.BlockSpec((1, D, F), lambda b, be, nu: (be[b], 0, 0)),
            pl.BlockSpec((1, D, F), lambda b, be, nu: (be[b], 0, 0)),
            pl.BlockSpec((1, F, D), lambda b, be, nu: (be[b], 0, 0)),
        ],
        out_specs=pl.BlockSpec((BT, D), lambda b, be, nu: (b, 0)),
    )
    return pl.pallas_call(
        _gmm_body,
        grid_spec=grid_spec,
        out_shape=jax.ShapeDtypeStruct((S, D), jnp.float32),
    )(be, nu, xs, w_gate, w_up, w_down)


@functools.partial(
    pl.kernel,
    out_type=jax.ShapeDtypeStruct((T, D), jnp.float32),
    mesh=_MESH,
    scratch_types=[
        pltpu.VMEM((CH,), jnp.int32),
        pltpu.VMEM((CH,), jnp.int32),
        pltpu.VMEM((CH, 16), jnp.float32),
        pltpu.VMEM((CH, 16), jnp.float32),
        pltpu.VMEM((CH, D), jnp.float32),
        pltpu.VMEM((CH, D), jnp.float32),
        pltpu.SemaphoreType.DMA,
        pltpu.SemaphoreType.DMA,
    ],
)
def _sc_combine(ys_hbm, d0_hbm, d1_hbm, w0_hbm, w1_hbm, out_hbm,
                i0_v, i1_v, w0_v, w1_v, buf0, buf1, sem0, sem1):
    wid = lax.axis_index("s") * NC + lax.axis_index("c")

    for c in range(TPW // CH):
        base = wid * TPW + c * CH
        pltpu.sync_copy(d0_hbm.at[pl.ds(base, CH)], i0_v)
        pltpu.sync_copy(d1_hbm.at[pl.ds(base, CH)], i1_v)
        cp0 = pltpu.async_copy(ys_hbm.at[i0_v], buf0, sem0)
        cp1 = pltpu.async_copy(ys_hbm.at[i1_v], buf1, sem1)
        pltpu.sync_copy(w0_hbm.at[pl.ds(base, CH)], w0_v)
        pltpu.sync_copy(w1_hbm.at[pl.ds(base, CH)], w1_v)
        cp0.wait()
        cp1.wait()

        def token_body(t, _):
            w0s = w0_v[t, :]
            w1s = w1_v[t, :]
            for v in range(D // 16):
                y0 = buf0[t, pl.ds(v * 16, 16)]
                y1 = buf1[t, pl.ds(v * 16, 16)]
                buf0[t, pl.ds(v * 16, 16)] = w0s * y0 + w1s * y1
            return 0

        lax.fori_loop(0, CH, token_body, 0)
        pltpu.sync_copy(buf0, out_hbm.at[pl.ds(base, CH)])


@jax.jit
def kernel(hidden_states, gate_w, w_gate, w_up, w_down):
    d0, d1, w0, w1, be, nu = _route(hidden_states, gate_w)
    d0r = d0.reshape(T)
    d1r = d1.reshape(T)
    xs = _sc_scatter(hidden_states, d0r, d1r)
    ys = _gmm(xs, w_gate, w_up, w_down, be.reshape(NB), nu.reshape(1))
    return _sc_combine(ys, d0r, d1r, w0, w1)


# speedup vs baseline: 1.4207x; 1.0275x over previous
import functools
import jax
import jax.numpy as jnp
from jax import lax
from jax.experimental import pallas as pl
from jax.experimental.pallas import tpu as pltpu
from jax.experimental.pallas import tpu_sc as plsc

E = 16
K = 2
D = 1024
F = 1024
T = 2048

BT = 256
NB = 32
S = NB * BT

NC = 2
NS = 16
NW = NC * NS
TPW = T // NW
CH = 16


def _route_body(x_ref, gw_ref, d0_ref, d1_ref, w0_ref, w1_ref, be_ref, nu_ref):
    x = x_ref[...]
    logits = jnp.dot(x, gw_ref[...], preferred_element_type=jnp.float32)
    m = jnp.max(logits, axis=-1, keepdims=True)
    ex = jnp.exp(logits - m)
    p = ex / jnp.sum(ex, axis=-1, keepdims=True)

    iota = lax.broadcasted_iota(jnp.int32, (T, E), 1)
    m0 = jnp.max(p, axis=-1, keepdims=True)
    i0 = jnp.min(jnp.where(p == m0, iota, E), axis=-1, keepdims=True)
    p_ex = jnp.where(iota == i0, -jnp.inf, p)
    m1 = jnp.max(p_ex, axis=-1, keepdims=True)
    i1 = jnp.min(jnp.where(p_ex == m1, iota, E), axis=-1, keepdims=True)
    wsum = m0 + m1
    w0_ref[...] = jnp.broadcast_to(m0 / wsum, (T, 16))
    w1_ref[...] = jnp.broadcast_to(m1 / wsum, (T, 16))

    onehot0 = (iota == i0).astype(jnp.float32)
    onehot1 = (iota == i1).astype(jnp.float32)
    cnt = onehot0 + onehot1

    n = jnp.sum(cnt, axis=0, keepdims=True)
    n_pad = jnp.floor((n + (BT - 1)) * (1.0 / BT)) * BT
    tri_e = (lax.broadcasted_iota(jnp.int32, (E, E), 0)
             < lax.broadcasted_iota(jnp.int32, (E, E), 1)).astype(jnp.float32)
    offs = jnp.dot(n_pad, tri_e, preferred_element_type=jnp.float32)
    ends = offs + n_pad

    bb = (lax.broadcasted_iota(jnp.int32, (NB, E), 0) * BT).astype(jnp.float32)
    ends_b = jnp.broadcast_to(ends, (NB, E))
    be = jnp.sum((bb >= ends_b).astype(jnp.float32), axis=-1, keepdims=True)
    be_ref[...] = jnp.minimum(be, float(E - 1)).astype(jnp.int32)
    nu_ref[...] = (jnp.sum(n_pad, axis=-1, keepdims=True)
                   * (1.0 / BT)).astype(jnp.int32)

    RC = 256
    NCK = T // RC
    t_iota = lax.broadcasted_iota(jnp.int32, (NCK, T), 1)
    c_iota = lax.broadcasted_iota(jnp.int32, (NCK, T), 0)
    sel = (t_iota // RC == c_iota).astype(jnp.float32)
    chunk_cnt = jnp.dot(sel, cnt, preferred_element_type=jnp.float32)
    tri_c = (lax.broadcasted_iota(jnp.int32, (NCK, NCK), 0)
             > lax.broadcasted_iota(jnp.int32, (NCK, NCK), 1)).astype(jnp.float32)
    excl_chunk = jnp.dot(tri_c, chunk_cnt, preferred_element_type=jnp.float32)

    row_l = lax.broadcasted_iota(jnp.int32, (RC, RC), 0)
    col_l = lax.broadcasted_iota(jnp.int32, (RC, RC), 1)
    tri_l = (col_l < row_l).astype(jnp.float32)
    for c in range(NCK):
        sl = slice(c * RC, (c + 1) * RC)
        rank = jnp.dot(tri_l, cnt[sl], preferred_element_type=jnp.float32)
        slot = jnp.broadcast_to(offs + excl_chunk[c:c + 1], (RC, E)) + rank
        d0 = jnp.sum(onehot0[sl] * slot, axis=-1, keepdims=True)
        d1 = jnp.sum(onehot1[sl] * slot, axis=-1, keepdims=True)
        d0_ref[sl] = d0.astype(jnp.int32)
        d1_ref[sl] = d1.astype(jnp.int32)


def _route(hidden_states, gate_w):
    return pl.pallas_call(
        _route_body,
        out_shape=[
            jax.ShapeDtypeStruct((T, 1), jnp.int32),
            jax.ShapeDtypeStruct((T, 1), jnp.int32),
            jax.ShapeDtypeStruct((T, 16), jnp.float32),
            jax.ShapeDtypeStruct((T, 16), jnp.float32),
            jax.ShapeDtypeStruct((NB, 1), jnp.int32),
            jax.ShapeDtypeStruct((1, 1), jnp.int32),
        ],
    )(hidden_states, gate_w)


_MESH = plsc.VectorSubcoreMesh(core_axis_name="c", subcore_axis_name="s")


@functools.partial(
    pl.kernel,
    out_type=jax.ShapeDtypeStruct((S, D), jnp.float32),
    mesh=_MESH,
    scratch_types=[
        pltpu.VMEM((TPW,), jnp.int32),
        pltpu.VMEM((TPW,), jnp.int32),
        pltpu.VMEM((TPW, D), jnp.float32),
        pltpu.SemaphoreType.DMA,
        pltpu.SemaphoreType.DMA,
    ],
)
def _sc_scatter(hid_hbm, d0_hbm, d1_hbm, xs_hbm, idx0_v, idx1_v, rows_v,
                sem_r, sem_s):
    wid = lax.axis_index("s") * NC + lax.axis_index("c")
    base = wid * TPW
    cp_r = pltpu.async_copy(hid_hbm.at[pl.ds(base, TPW)], rows_v, sem_r)
    pltpu.sync_copy(d0_hbm.at[pl.ds(base, TPW)], idx0_v)
    pltpu.sync_copy(d1_hbm.at[pl.ds(base, TPW)], idx1_v)
    cp_r.wait()
    cp0 = pltpu.async_copy(rows_v, xs_hbm.at[idx0_v], sem_s)
    cp1 = pltpu.async_copy(rows_v, xs_hbm.at[idx1_v], sem_s)
    cp0.wait()
    cp1.wait()


def _gmm_body(be_ref, nu_ref, xs_ref, wg_ref, wu_ref, wd_ref, ys_ref):
    b = pl.program_id(0)

    @pl.when(b < nu_ref[0])
    def _():
        x = xs_ref[...]
        h = jax.nn.silu(jnp.dot(x, wg_ref[0], preferred_element_type=jnp.float32))
        h = h * jnp.dot(x, wu_ref[0], preferred_element_type=jnp.float32)
        ys_ref[...] = jnp.dot(h, wd_ref[0], preferred_element_type=jnp.float32)


def _gmm(xs, w_gate, w_up, w_down, be, nu):
    grid_spec = pltpu.PrefetchScalarGridSpec(
        num_scalar_prefetch=2,
        grid=(NB,),
        in_specs=[
            pl.BlockSpec((BT, D), lambda b, be, nu: (b, 0)),
            pl.BlockSpec((1, D, F), lambda b, be, nu: (be[b], 0, 0)),
            pl.BlockSpec((1, D, F), lambda b, be, nu: (be[b], 0, 0)),
            pl.BlockSpec((1, F, D), lambda b, be, nu: (be[b], 0, 0)),
        ],
        out_specs=pl.BlockSpec((BT, D), lambda b, be, nu: (b, 0)),
    )
    return pl.pallas_call(
        _gmm_body,
        grid_spec=grid_spec,
        out_shape=jax.ShapeDtypeStruct((S, D), jnp.float32),
    )(be, nu, xs, w_gate, w_up, w_down)


NCH = TPW // CH


@functools.partial(
    pl.kernel,
    out_type=jax.ShapeDtypeStruct((T, D), jnp.float32),
    mesh=_MESH,
    scratch_types=[
        pltpu.VMEM((TPW,), jnp.int32),
        pltpu.VMEM((TPW,), jnp.int32),
        pltpu.VMEM((TPW, 16), jnp.float32),
        pltpu.VMEM((TPW, 16), jnp.float32),
        [pltpu.VMEM((CH, D), jnp.float32)] * 2,
        [pltpu.VMEM((CH, D), jnp.float32)] * 2,
        [pltpu.VMEM((CH, D), jnp.float32)] * 2,
        [pltpu.SemaphoreType.DMA] * 2,
        [pltpu.SemaphoreType.DMA] * 2,
    ],
)
def _sc_combine(ys_hbm, d0_hbm, d1_hbm, w0_hbm, w1_hbm, out_hbm,
                i0_v, i1_v, w0_v, w1_v, y0b, y1b, ob, gsem, osem):
    wid = lax.axis_index("s") * NC + lax.axis_index("c")
    base = wid * TPW
    pltpu.sync_copy(d0_hbm.at[pl.ds(base, TPW)], i0_v)
    pltpu.sync_copy(d1_hbm.at[pl.ds(base, TPW)], i1_v)
    pltpu.sync_copy(w0_hbm.at[pl.ds(base, TPW)], w0_v)
    pltpu.sync_copy(w1_hbm.at[pl.ds(base, TPW)], w1_v)

    def start(c):
        q = c % 2
        sl = pl.ds(c * CH, CH)
        return (pltpu.async_copy(ys_hbm.at[i0_v.at[sl]], y0b[q], gsem[q]),
                pltpu.async_copy(ys_hbm.at[i1_v.at[sl]], y1b[q], gsem[q]))

    inflight = start(0)
    out_cps = [None, None]
    for c in range(NCH):
        q = c % 2
        for cp in inflight:
            cp.wait()
        if c + 1 < NCH:
            inflight = start(c + 1)
        if out_cps[q] is not None:
            out_cps[q].wait()

        def token_body(t, _):
            w0s = w0_v[c * CH + t, :]
            w1s = w1_v[c * CH + t, :]
            for v in range(D // 16):
                y0 = y0b[q][t, pl.ds(v * 16, 16)]
                y1 = y1b[q][t, pl.ds(v * 16, 16)]
                ob[q][t, pl.ds(v * 16, 16)] = w0s * y0 + w1s * y1
            return 0

        lax.fori_loop(0, CH, token_body, 0)
        out_cps[q] = pltpu.async_copy(
            ob[q], out_hbm.at[pl.ds(base + c * CH, CH)], osem[q])
    for cp in out_cps:
        if cp is not None:
            cp.wait()


@jax.jit
def kernel(hidden_states, gate_w, w_gate, w_up, w_down):
    d0, d1, w0, w1, be, nu = _route(hidden_states, gate_w)
    d0r = d0.reshape(T)
    d1r = d1.reshape(T)
    xs = _sc_scatter(hidden_states, d0r, d1r)
    ys = _gmm(xs, w_gate, w_up, w_down, be.reshape(NB), nu.reshape(1))
    return _sc_combine(ys, d0r, d1r, w0, w1)


# speedup vs baseline: 1.4880x; 1.0473x over previous
import functools
import jax
import jax.numpy as jnp
from jax import lax
from jax.experimental import pallas as pl
from jax.experimental.pallas import tpu as pltpu
from jax.experimental.pallas import tpu_sc as plsc

E = 16
K = 2
D = 1024
F = 1024
T = 2048

BT = 256
NB = 32
S = NB * BT

NC = 2
NS = 16
NW = NC * NS
TPW = T // NW
CH = 16


def _route_body(x_ref, gw_ref, d0_ref, d1_ref, w0_ref, w1_ref, be_ref, nu_ref):
    x = x_ref[...]
    logits = jnp.dot(x, gw_ref[...], preferred_element_type=jnp.float32)
    m = jnp.max(logits, axis=-1, keepdims=True)
    ex = jnp.exp(logits - m)
    p = ex / jnp.sum(ex, axis=-1, keepdims=True)

    iota = lax.broadcasted_iota(jnp.int32, (T, E), 1)
    m0 = jnp.max(p, axis=-1, keepdims=True)
    i0 = jnp.min(jnp.where(p == m0, iota, E), axis=-1, keepdims=True)
    p_ex = jnp.where(iota == i0, -jnp.inf, p)
    m1 = jnp.max(p_ex, axis=-1, keepdims=True)
    i1 = jnp.min(jnp.where(p_ex == m1, iota, E), axis=-1, keepdims=True)
    wsum = m0 + m1
    w0_ref[...] = jnp.broadcast_to(m0 / wsum, (T, 16))
    w1_ref[...] = jnp.broadcast_to(m1 / wsum, (T, 16))

    onehot0 = (iota == i0).astype(jnp.float32)
    onehot1 = (iota == i1).astype(jnp.float32)
    cnt = onehot0 + onehot1

    n = jnp.sum(cnt, axis=0, keepdims=True)
    n_pad = jnp.floor((n + (BT - 1)) * (1.0 / BT)) * BT
    tri_e = (lax.broadcasted_iota(jnp.int32, (E, E), 0)
             < lax.broadcasted_iota(jnp.int32, (E, E), 1)).astype(jnp.float32)
    offs = jnp.dot(n_pad, tri_e, preferred_element_type=jnp.float32)
    ends = offs + n_pad

    bb = (lax.broadcasted_iota(jnp.int32, (NB, E), 0) * BT).astype(jnp.float32)
    ends_b = jnp.broadcast_to(ends, (NB, E))
    be = jnp.sum((bb >= ends_b).astype(jnp.float32), axis=-1, keepdims=True)
    be_ref[...] = jnp.minimum(be, float(E - 1)).astype(jnp.int32)
    nu_ref[...] = (jnp.sum(n_pad, axis=-1, keepdims=True)
                   * (1.0 / BT)).astype(jnp.int32)

    RC = 256
    NCK = T // RC
    t_iota = lax.broadcasted_iota(jnp.int32, (NCK, T), 1)
    c_iota = lax.broadcasted_iota(jnp.int32, (NCK, T), 0)
    sel = (t_iota // RC == c_iota).astype(jnp.float32)
    chunk_cnt = jnp.dot(sel, cnt, preferred_element_type=jnp.float32)
    tri_c = (lax.broadcasted_iota(jnp.int32, (NCK, NCK), 0)
             > lax.broadcasted_iota(jnp.int32, (NCK, NCK), 1)).astype(jnp.float32)
    excl_chunk = jnp.dot(tri_c, chunk_cnt, preferred_element_type=jnp.float32)

    row_l = lax.broadcasted_iota(jnp.int32, (RC, RC), 0)
    col_l = lax.broadcasted_iota(jnp.int32, (RC, RC), 1)
    tri_l = (col_l < row_l).astype(jnp.float32)
    for c in range(NCK):
        sl = slice(c * RC, (c + 1) * RC)
        rank = jnp.dot(tri_l, cnt[sl], preferred_element_type=jnp.float32)
        slot = jnp.broadcast_to(offs + excl_chunk[c:c + 1], (RC, E)) + rank
        d0 = jnp.sum(onehot0[sl] * slot, axis=-1, keepdims=True)
        d1 = jnp.sum(onehot1[sl] * slot, axis=-1, keepdims=True)
        d0_ref[sl] = d0.astype(jnp.int32)
        d1_ref[sl] = d1.astype(jnp.int32)


def _route(hidden_states, gate_w):
    return pl.pallas_call(
        _route_body,
        out_shape=[
            jax.ShapeDtypeStruct((T, 1), jnp.int32),
            jax.ShapeDtypeStruct((T, 1), jnp.int32),
            jax.ShapeDtypeStruct((T, 16), jnp.float32),
            jax.ShapeDtypeStruct((T, 16), jnp.float32),
            jax.ShapeDtypeStruct((NB, 1), jnp.int32),
            jax.ShapeDtypeStruct((1, 1), jnp.int32),
        ],
    )(hidden_states, gate_w)


_MESH = plsc.VectorSubcoreMesh(core_axis_name="c", subcore_axis_name="s")


@functools.partial(
    pl.kernel,
    out_type=jax.ShapeDtypeStruct((S, D), jnp.float32),
    mesh=_MESH,
    scratch_types=[
        pltpu.VMEM((TPW,), jnp.int32),
        pltpu.VMEM((TPW,), jnp.int32),
        pltpu.VMEM((TPW, D), jnp.float32),
        pltpu.SemaphoreType.DMA,
        pltpu.SemaphoreType.DMA,
    ],
)
def _sc_scatter(hid_hbm, d0_hbm, d1_hbm, xs_hbm, idx0_v, idx1_v, rows_v,
                sem_r, sem_s):
    wid = lax.axis_index("s") * NC + lax.axis_index("c")
    base = wid * TPW
    cp_r = pltpu.async_copy(hid_hbm.at[pl.ds(base, TPW)], rows_v, sem_r)
    pltpu.sync_copy(d0_hbm.at[pl.ds(base, TPW)], idx0_v)
    pltpu.sync_copy(d1_hbm.at[pl.ds(base, TPW)], idx1_v)
    cp_r.wait()
    cp0 = pltpu.async_copy(rows_v, xs_hbm.at[idx0_v], sem_s)
    cp1 = pltpu.async_copy(rows_v, xs_hbm.at[idx1_v], sem_s)
    cp0.wait()
    cp1.wait()


def _gmm_body(be_ref, nu_ref, xs_ref, wg_ref, wu_ref, wd_ref, ys_ref):
    b = pl.program_id(0)

    @pl.when(b < nu_ref[0])
    def _():
        x = xs_ref[...]
        h = jax.nn.silu(jnp.dot(x, wg_ref[0], preferred_element_type=jnp.float32))
        h = h * jnp.dot(x, wu_ref[0], preferred_element_type=jnp.float32)
        ys_ref[...] = jnp.dot(h, wd_ref[0], preferred_element_type=jnp.float32)


def _gmm(xs, w_gate, w_up, w_down, be, nu):
    grid_spec = pltpu.PrefetchScalarGridSpec(
        num_scalar_prefetch=2,
        grid=(nu[0],),
        in_specs=[
            pl.BlockSpec((BT, D), lambda b, be, nu: (b, 0)),
            pl.BlockSpec((1, D, F), lambda b, be, nu: (be[b], 0, 0)),
            pl.BlockSpec((1, D, F), lambda b, be, nu: (be[b], 0, 0)),
            pl.BlockSpec((1, F, D), lambda b, be, nu: (be[b], 0, 0)),
        ],
        out_specs=pl.BlockSpec((BT, D), lambda b, be, nu: (b, 0)),
    )
    return pl.pallas_call(
        _gmm_body,
        grid_spec=grid_spec,
        out_shape=jax.ShapeDtypeStruct((S, D), jnp.float32),
    )(be, nu, xs, w_gate, w_up, w_down)


NCH = TPW // CH


@functools.partial(
    pl.kernel,
    out_type=jax.ShapeDtypeStruct((T, D), jnp.float32),
    mesh=_MESH,
    scratch_types=[
        pltpu.VMEM((TPW,), jnp.int32),
        pltpu.VMEM((TPW,), jnp.int32),
        pltpu.VMEM((TPW, 16), jnp.float32),
        pltpu.VMEM((TPW, 16), jnp.float32),
        [pltpu.VMEM((CH, D), jnp.float32)] * 2,
        [pltpu.VMEM((CH, D), jnp.float32)] * 2,
        [pltpu.VMEM((CH, D), jnp.float32)] * 2,
        [pltpu.SemaphoreType.DMA] * 2,
        [pltpu.SemaphoreType.DMA] * 2,
    ],
)
def _sc_combine(ys_hbm, d0_hbm, d1_hbm, w0_hbm, w1_hbm, out_hbm,
                i0_v, i1_v, w0_v, w1_v, y0b, y1b, ob, gsem, osem):
    wid = lax.axis_index("s") * NC + lax.axis_index("c")
    base = wid * TPW
    pltpu.sync_copy(d0_hbm.at[pl.ds(base, TPW)], i0_v)
    pltpu.sync_copy(d1_hbm.at[pl.ds(base, TPW)], i1_v)
    pltpu.sync_copy(w0_hbm.at[pl.ds(base, TPW)], w0_v)
    pltpu.sync_copy(w1_hbm.at[pl.ds(base, TPW)], w1_v)

    def start(c):
        q = c % 2
        sl = pl.ds(c * CH, CH)
        return (pltpu.async_copy(ys_hbm.at[i0_v.at[sl]], y0b[q], gsem[q]),
                pltpu.async_copy(ys_hbm.at[i1_v.at[sl]], y1b[q], gsem[q]))

    inflight = start(0)
    out_cps = [None, None]
    for c in range(NCH):
        q = c % 2
        for cp in inflight:
            cp.wait()
        if c + 1 < NCH:
            inflight = start(c + 1)
        if out_cps[q] is not None:
            out_cps[q].wait()

        def token_body(t, _):
            w0s = w0_v[c * CH + t, :]
            w1s = w1_v[c * CH + t, :]
            for v in range(D // 16):
                y0 = y0b[q][t, pl.ds(v * 16, 16)]
                y1 = y1b[q][t, pl.ds(v * 16, 16)]
                ob[q][t, pl.ds(v * 16, 16)] = w0s * y0 + w1s * y1
            return 0

        lax.fori_loop(0, CH, token_body, 0)
        out_cps[q] = pltpu.async_copy(
            ob[q], out_hbm.at[pl.ds(base + c * CH, CH)], osem[q])
    for cp in out_cps:
        if cp is not None:
            cp.wait()


@jax.jit
def kernel(hidden_states, gate_w, w_gate, w_up, w_down):
    d0, d1, w0, w1, be, nu = _route(hidden_states, gate_w)
    d0r = d0.reshape(T)
    d1r = d1.reshape(T)
    xs = _sc_scatter(hidden_states, d0r, d1r)
    ys = _gmm(xs, w_gate, w_up, w_down, be.reshape(NB), nu.reshape(1))
    return _sc_combine(ys, d0r, d1r, w0, w1)


# speedup vs baseline: 1.5113x; 1.0157x over previous
import functools
import jax
import jax.numpy as jnp
from jax import lax
from jax.experimental import pallas as pl
from jax.experimental.pallas import tpu as pltpu
from jax.experimental.pallas import tpu_sc as plsc

E = 16
K = 2
D = 1024
F = 1024
T = 2048

BT = 256
NB = 32
S = NB * BT

NC = 2
NS = 16
NW = NC * NS
TPW = T // NW
CH = 16


def _route_body(x_ref, gw_ref, d0_ref, d1_ref, w0_ref, w1_ref, be_ref, nu_ref):
    x = x_ref[...]
    logits = jnp.dot(x, gw_ref[...], preferred_element_type=jnp.float32)
    m = jnp.max(logits, axis=-1, keepdims=True)
    ex = jnp.exp(logits - m)
    p = ex / jnp.sum(ex, axis=-1, keepdims=True)

    iota = lax.broadcasted_iota(jnp.int32, (T, E), 1)
    m0 = jnp.max(p, axis=-1, keepdims=True)
    i0 = jnp.min(jnp.where(p == m0, iota, E), axis=-1, keepdims=True)
    p_ex = jnp.where(iota == i0, -jnp.inf, p)
    m1 = jnp.max(p_ex, axis=-1, keepdims=True)
    i1 = jnp.min(jnp.where(p_ex == m1, iota, E), axis=-1, keepdims=True)
    wsum = m0 + m1
    w0_ref[...] = jnp.broadcast_to(m0 / wsum, (T, 16))
    w1_ref[...] = jnp.broadcast_to(m1 / wsum, (T, 16))

    onehot0 = (iota == i0).astype(jnp.float32)
    onehot1 = (iota == i1).astype(jnp.float32)
    cnt = onehot0 + onehot1

    n = jnp.sum(cnt, axis=0, keepdims=True)
    n_pad = jnp.floor((n + (BT - 1)) * (1.0 / BT)) * BT
    tri_e = (lax.broadcasted_iota(jnp.int32, (E, E), 0)
             < lax.broadcasted_iota(jnp.int32, (E, E), 1)).astype(jnp.float32)
    offs = jnp.dot(n_pad, tri_e, preferred_element_type=jnp.float32)
    ends = offs + n_pad

    bb = (lax.broadcasted_iota(jnp.int32, (NB, E), 0) * BT).astype(jnp.float32)
    ends_b = jnp.broadcast_to(ends, (NB, E))
    be = jnp.sum((bb >= ends_b).astype(jnp.float32), axis=-1, keepdims=True)
    be_ref[...] = jnp.minimum(be, float(E - 1)).astype(jnp.int32)
    nu_ref[...] = (jnp.sum(n_pad, axis=-1, keepdims=True)
                   * (1.0 / BT)).astype(jnp.int32)

    RC = 256
    NCK = T // RC
    t_iota = lax.broadcasted_iota(jnp.int32, (NCK, T), 1)
    c_iota = lax.broadcasted_iota(jnp.int32, (NCK, T), 0)
    sel = (t_iota // RC == c_iota).astype(jnp.float32)
    chunk_cnt = jnp.dot(sel, cnt, preferred_element_type=jnp.float32)
    tri_c = (lax.broadcasted_iota(jnp.int32, (NCK, NCK), 0)
             > lax.broadcasted_iota(jnp.int32, (NCK, NCK), 1)).astype(jnp.float32)
    excl_chunk = jnp.dot(tri_c, chunk_cnt, preferred_element_type=jnp.float32)

    row_l = lax.broadcasted_iota(jnp.int32, (RC, RC), 0)
    col_l = lax.broadcasted_iota(jnp.int32, (RC, RC), 1)
    tri_l = (col_l < row_l).astype(jnp.float32)
    for c in range(NCK):
        sl = slice(c * RC, (c + 1) * RC)
        rank = jnp.dot(tri_l, cnt[sl], preferred_element_type=jnp.float32)
        slot = jnp.broadcast_to(offs + excl_chunk[c:c + 1], (RC, E)) + rank
        d0 = jnp.sum(onehot0[sl] * slot, axis=-1, keepdims=True)
        d1 = jnp.sum(onehot1[sl] * slot, axis=-1, keepdims=True)
        d0_ref[sl] = d0.astype(jnp.int32)
        d1_ref[sl] = d1.astype(jnp.int32)


def _route(hidden_states, gate_w):
    return pl.pallas_call(
        _route_body,
        out_shape=[
            jax.ShapeDtypeStruct((T, 1), jnp.int32),
            jax.ShapeDtypeStruct((T, 1), jnp.int32),
            jax.ShapeDtypeStruct((T, 16), jnp.float32),
            jax.ShapeDtypeStruct((T, 16), jnp.float32),
            jax.ShapeDtypeStruct((NB, 1), jnp.int32),
            jax.ShapeDtypeStruct((1, 1), jnp.int32),
        ],
    )(hidden_states, gate_w)


_MESH = plsc.VectorSubcoreMesh(core_axis_name="c", subcore_axis_name="s")


@functools.partial(
    pl.kernel,
    out_type=jax.ShapeDtypeStruct((S, D), jnp.float32),
    mesh=_MESH,
    scratch_types=[
        [pltpu.VMEM((TPW // 2,), jnp.int32)] * 2,
        [pltpu.VMEM((TPW // 2,), jnp.int32)] * 2,
        [pltpu.VMEM((TPW // 2, D), jnp.float32)] * 2,
        pltpu.SemaphoreType.DMA,
        pltpu.SemaphoreType.DMA,
    ],
)
def _sc_scatter(hid_hbm, d0_hbm, d1_hbm, xs_hbm, idx0_v, idx1_v, rows_v,
                sem_r, sem_s):
    wid = lax.axis_index("s") * NC + lax.axis_index("c")
    base = wid * TPW
    HH = TPW // 2
    row_cps = [pltpu.async_copy(hid_hbm.at[pl.ds(base + h * HH, HH)],
                                rows_v[h], sem_r) for h in range(2)]
    for h in range(2):
        pltpu.sync_copy(d0_hbm.at[pl.ds(base + h * HH, HH)], idx0_v[h])
        pltpu.sync_copy(d1_hbm.at[pl.ds(base + h * HH, HH)], idx1_v[h])
    scat_cps = []
    for h in range(2):
        row_cps[h].wait()
        scat_cps.append(pltpu.async_copy(rows_v[h], xs_hbm.at[idx0_v[h]], sem_s))
        scat_cps.append(pltpu.async_copy(rows_v[h], xs_hbm.at[idx1_v[h]], sem_s))
    for cp in scat_cps:
        cp.wait()


def _gmm_body(be_ref, nu_ref, xs_ref, wg_ref, wu_ref, wd_ref, ys_ref):
    b = pl.program_id(0)

    @pl.when(b < nu_ref[0])
    def _():
        x = xs_ref[...]
        h = jax.nn.silu(jnp.dot(x, wg_ref[0], preferred_element_type=jnp.float32))
        h = h * jnp.dot(x, wu_ref[0], preferred_element_type=jnp.float32)
        ys_ref[...] = jnp.dot(h, wd_ref[0], preferred_element_type=jnp.float32)


def _gmm(xs, w_gate, w_up, w_down, be, nu):
    grid_spec = pltpu.PrefetchScalarGridSpec(
        num_scalar_prefetch=2,
        grid=(nu[0],),
        in_specs=[
            pl.BlockSpec((BT, D), lambda b, be, nu: (b, 0)),
            pl.BlockSpec((1, D, F), lambda b, be, nu: (be[b], 0, 0)),
            pl.BlockSpec((1, D, F), lambda b, be, nu: (be[b], 0, 0)),
            pl.BlockSpec((1, F, D), lambda b, be, nu: (be[b], 0, 0)),
        ],
        out_specs=pl.BlockSpec((BT, D), lambda b, be, nu: (b, 0)),
    )
    return pl.pallas_call(
        _gmm_body,
        grid_spec=grid_spec,
        out_shape=jax.ShapeDtypeStruct((S, D), jnp.float32),
    )(be, nu, xs, w_gate, w_up, w_down)


NCH = TPW // CH


@functools.partial(
    pl.kernel,
    out_type=jax.ShapeDtypeStruct((T, D), jnp.float32),
    mesh=_MESH,
    scratch_types=[
        pltpu.VMEM((TPW,), jnp.int32),
        pltpu.VMEM((TPW,), jnp.int32),
        pltpu.VMEM((TPW, 16), jnp.float32),
        pltpu.VMEM((TPW, 16), jnp.float32),
        [pltpu.VMEM((CH, D), jnp.float32)] * 2,
        [pltpu.VMEM((CH, D), jnp.float32)] * 2,
        [pltpu.VMEM((CH, D), jnp.float32)] * 2,
        [pltpu.SemaphoreType.DMA] * 2,
        [pltpu.SemaphoreType.DMA] * 2,
    ],
)
def _sc_combine(ys_hbm, d0_hbm, d1_hbm, w0_hbm, w1_hbm, out_hbm,
                i0_v, i1_v, w0_v, w1_v, y0b, y1b, ob, gsem, osem):
    wid = lax.axis_index("s") * NC + lax.axis_index("c")
    base = wid * TPW
    pltpu.sync_copy(d0_hbm.at[pl.ds(base, TPW)], i0_v)
    pltpu.sync_copy(d1_hbm.at[pl.ds(base, TPW)], i1_v)
    w_cps = [pltpu.async_copy(w0_hbm.at[pl.ds(base, TPW)], w0_v, osem[0]),
             pltpu.async_copy(w1_hbm.at[pl.ds(base, TPW)], w1_v, osem[1])]

    def start(c):
        q = c % 2
        sl = pl.ds(c * CH, CH)
        return (pltpu.async_copy(ys_hbm.at[i0_v.at[sl]], y0b[q], gsem[q]),
                pltpu.async_copy(ys_hbm.at[i1_v.at[sl]], y1b[q], gsem[q]))

    inflight = start(0)
    out_cps = [None, None]
    for c in range(NCH):
        q = c % 2
        for cp in inflight:
            cp.wait()
        if c + 1 < NCH:
            inflight = start(c + 1)
        if c == 0:
            for cp in w_cps:
                cp.wait()
        if out_cps[q] is not None:
            out_cps[q].wait()

        def token_body(t, _):
            w0s = w0_v[c * CH + t, :]
            w1s = w1_v[c * CH + t, :]
            for v in range(D // 16):
                y0 = y0b[q][t, pl.ds(v * 16, 16)]
                y1 = y1b[q][t, pl.ds(v * 16, 16)]
                ob[q][t, pl.ds(v * 16, 16)] = w0s * y0 + w1s * y1
            return 0

        lax.fori_loop(0, CH, token_body, 0)
        out_cps[q] = pltpu.async_copy(
            ob[q], out_hbm.at[pl.ds(base + c * CH, CH)], osem[q])
    for cp in out_cps:
        if cp is not None:
            cp.wait()


@jax.jit
def kernel(hidden_states, gate_w, w_gate, w_up, w_down):
    d0, d1, w0, w1, be, nu = _route(hidden_states, gate_w)
    d0r = d0.reshape(T)
    d1r = d1.reshape(T)
    xs = _sc_scatter(hidden_states, d0r, d1r)
    ys = _gmm(xs, w_gate, w_up, w_down, be.reshape(NB), nu.reshape(1))
    return _sc_combine(ys, d0r, d1r, w0, w1)
